# baseline (device time: 29126 ns/iter reference)
import jax
import jax.numpy as jnp
from jax import lax
from jax.experimental import pallas as pl
from jax.experimental.pallas import tpu as pltpu

N_DEV = 4


def kernel(A, B):
    m, k = A.shape
    _, n = B.shape

    def body(a_ref, b_ref, out_ref, comm_ref, send_sems, recv_sems):
        my_pos = lax.axis_index("i")
        left = (my_pos - 1) % N_DEV
        right = (my_pos + 1) % N_DEV

        barrier_sem = pltpu.get_barrier_semaphore()
        for nbr in [left, right]:
            pl.semaphore_signal(
                barrier_sem, inc=1,
                device_id=(nbr,), device_id_type=pl.DeviceIdType.MESH,
            )
        pl.semaphore_wait(barrier_sem, 2)

        partial = jnp.dot(
            a_ref[...].astype(jnp.bfloat16),
            b_ref[...].astype(jnp.bfloat16),
            preferred_element_type=jnp.float32,
        )
        out_ref[...] = partial
        comm_ref[0] = partial.astype(jnp.bfloat16)

        for h in range(N_DEV - 1):
            send_slot = h % 2
            recv_slot = (h + 1) % 2
            rdma = pltpu.make_async_remote_copy(
                src_ref=comm_ref.at[send_slot],
                dst_ref=comm_ref.at[recv_slot],
                send_sem=send_sems.at[send_slot],
                recv_sem=recv_sems.at[recv_slot],
                device_id=(right,),
                device_id_type=pl.DeviceIdType.MESH,
            )
            rdma.start()
            rdma.wait()
            out_ref[...] += comm_ref[recv_slot].astype(jnp.float32)

        z = out_ref[...]
        out_ref[...] = z / (1.0 + jnp.exp(-z))

    return pl.pallas_call(
        body,
        out_shape=jax.ShapeDtypeStruct((m, n), jnp.float32),
        in_specs=[
            pl.BlockSpec(memory_space=pltpu.VMEM),
            pl.BlockSpec(memory_space=pltpu.VMEM),
        ],
        out_specs=pl.BlockSpec(memory_space=pltpu.VMEM),
        scratch_shapes=[
            pltpu.VMEM((2, m, n), jnp.bfloat16),
            pltpu.SemaphoreType.DMA((2,)),
            pltpu.SemaphoreType.DMA((2,)),
        ],
        compiler_params=pltpu.CompilerParams(collective_id=0),
    )(A, B)


# device time: 15671 ns/iter; 1.8586x vs baseline; 1.8586x over previous
import jax
import jax.numpy as jnp
from jax import lax
from jax.experimental import pallas as pl
from jax.experimental.pallas import tpu as pltpu

N_DEV = 4


def kernel(A, B):
    m, k = A.shape
    _, n = B.shape
    half = m // 2

    def body(a_ref, b_ref, out_ref, stage_ref, comm_ref, send_sems, recv_sems):
        my_pos = lax.axis_index("i")
        left = (my_pos - 1) % N_DEV
        right = (my_pos + 1) % N_DEV
        pa = my_pos ^ 1
        pb = 3 - my_pos

        barrier_sem = pltpu.get_barrier_semaphore()
        for nbr in [left, right]:
            pl.semaphore_signal(
                barrier_sem, inc=1,
                device_id=(nbr,), device_id_type=pl.DeviceIdType.MESH,
            )
        pl.semaphore_wait(barrier_sem, 2)

        partial = jnp.dot(
            a_ref[...].astype(jnp.bfloat16),
            b_ref[...].astype(jnp.bfloat16),
            preferred_element_type=jnp.float32,
        )
        pb16 = partial.astype(jnp.bfloat16)
        stage_ref[0, 0] = pb16[:half]
        stage_ref[0, 1] = pb16[half:]

        def xfer(phase, slot, target):
            return pltpu.make_async_remote_copy(
                src_ref=stage_ref.at[phase, slot],
                dst_ref=comm_ref.at[phase, slot],
                send_sem=send_sems.at[phase, slot],
                recv_sem=recv_sems.at[phase, slot],
                device_id=(target,),
                device_id_type=pl.DeviceIdType.MESH,
            )

        r1a = xfer(0, 0, pa)
        r1b = xfer(0, 1, pb)
        r1a.start()
        r1b.start()
        r1a.wait()
        r1b.wait()

        top_red = partial[:half] + comm_ref[0, 0].astype(jnp.float32)
        bot_red = partial[half:] + comm_ref[0, 1].astype(jnp.float32)
        stage_ref[1, 0] = top_red.astype(jnp.bfloat16)
        stage_ref[1, 1] = bot_red.astype(jnp.bfloat16)

        r2a = xfer(1, 0, pb)
        r2b = xfer(1, 1, pa)
        r2a.start()
        r2b.start()
        r2a.wait()
        r2b.wait()

        top = top_red + comm_ref[1, 0].astype(jnp.float32)
        bot = bot_red + comm_ref[1, 1].astype(jnp.float32)
        out_ref[:half] = top / (1.0 + jnp.exp(-top))
        out_ref[half:] = bot / (1.0 + jnp.exp(-bot))

    return pl.pallas_call(
        body,
        out_shape=jax.ShapeDtypeStruct((m, n), jnp.float32),
        in_specs=[
            pl.BlockSpec(memory_space=pltpu.VMEM),
            pl.BlockSpec(memory_space=pltpu.VMEM),
        ],
        out_specs=pl.BlockSpec(memory_space=pltpu.VMEM),
        scratch_shapes=[
            pltpu.VMEM((2, 2, half, n), jnp.bfloat16),
            pltpu.VMEM((2, 2, half, n), jnp.bfloat16),
            pltpu.SemaphoreType.DMA((2, 2)),
            pltpu.SemaphoreType.DMA((2, 2)),
        ],
        compiler_params=pltpu.CompilerParams(collective_id=0),
    )(A, B)


# device time: 15523 ns/iter; 1.8763x vs baseline; 1.0095x over previous
import jax
import jax.numpy as jnp
from jax import lax
from jax.experimental import pallas as pl
from jax.experimental.pallas import tpu as pltpu

N_DEV = 4


def kernel(A, B):
    m, k = A.shape
    _, n = B.shape
    half = m // 2

    def body(a_ref, b_ref, out_ref, stage_ref, comm_ref, send_sems, recv_sems):
        my_pos = lax.axis_index("i")
        left = (my_pos - 1) % N_DEV
        right = (my_pos + 1) % N_DEV
        pa = my_pos ^ 1
        pb = 3 - my_pos

        barrier_sem = pltpu.get_barrier_semaphore()
        for nbr in [left, right]:
            pl.semaphore_signal(
                barrier_sem, inc=1,
                device_id=(nbr,), device_id_type=pl.DeviceIdType.MESH,
            )
        pl.semaphore_wait(barrier_sem, 2)

        def xfer(phase, slot, target):
            return pltpu.make_async_remote_copy(
                src_ref=stage_ref.at[phase, slot],
                dst_ref=comm_ref.at[phase, slot],
                send_sem=send_sems.at[phase, slot],
                recv_sem=recv_sems.at[phase, slot],
                device_id=(target,),
                device_id_type=pl.DeviceIdType.MESH,
            )

        r1a = xfer(0, 0, pa)
        r1b = xfer(0, 1, pb)
        r2a = xfer(1, 0, pb)
        r2b = xfer(1, 1, pa)

        a16 = a_ref[...].astype(jnp.bfloat16)
        b16 = b_ref[...].astype(jnp.bfloat16)

        p_top = jnp.dot(a16[:half], b16, preferred_element_type=jnp.float32)
        stage_ref[0, 0] = p_top.astype(jnp.bfloat16)
        r1a.start()
        p_bot = jnp.dot(a16[half:], b16, preferred_element_type=jnp.float32)
        stage_ref[0, 1] = p_bot.astype(jnp.bfloat16)
        r1b.start()

        r1a.wait_recv()
        top_red = p_top + comm_ref[0, 0].astype(jnp.float32)
        stage_ref[1, 0] = top_red.astype(jnp.bfloat16)
        r2a.start()
        r1b.wait_recv()
        bot_red = p_bot + comm_ref[0, 1].astype(jnp.float32)
        stage_ref[1, 1] = bot_red.astype(jnp.bfloat16)
        r2b.start()

        r2a.wait_recv()
        top = top_red + comm_ref[1, 0].astype(jnp.float32)
        out_ref[:half] = top / (1.0 + jnp.exp(-top))
        r2b.wait_recv()
        bot = bot_red + comm_ref[1, 1].astype(jnp.float32)
        out_ref[half:] = bot / (1.0 + jnp.exp(-bot))

        r1a.wait_send()
        r1b.wait_send()
        r2a.wait_send()
        r2b.wait_send()

    return pl.pallas_call(
        body,
        out_shape=jax.ShapeDtypeStruct((m, n), jnp.float32),
        in_specs=[
            pl.BlockSpec(memory_space=pltpu.VMEM),
            pl.BlockSpec(memory_space=pltpu.VMEM),
        ],
        out_specs=pl.BlockSpec(memory_space=pltpu.VMEM),
        scratch_shapes=[
            pltpu.VMEM((2, 2, half, n), jnp.bfloat16),
            pltpu.VMEM((2, 2, half, n), jnp.bfloat16),
            pltpu.SemaphoreType.DMA((2, 2)),
            pltpu.SemaphoreType.DMA((2, 2)),
        ],
        compiler_params=pltpu.CompilerParams(collective_id=0),
    )(A, B)


# device time: 14205 ns/iter; 2.0504x vs baseline; 1.0928x over previous
import jax
import jax.numpy as jnp
from jax import lax
from jax.experimental import pallas as pl
from jax.experimental.pallas import tpu as pltpu

N_DEV = 4
Q = 4


def kernel(A, B):
    m, k = A.shape
    _, n = B.shape
    rows = m // Q

    def body(a_ref, b_ref, out_ref, stage_ref, comm_ref, send_sems, recv_sems):
        my_pos = lax.axis_index("i")
        left = (my_pos - 1) % N_DEV
        right = (my_pos + 1) % N_DEV
        pa = my_pos ^ 1
        pb = 3 - my_pos

        barrier_sem = pltpu.get_barrier_semaphore()
        for nbr in [left, right]:
            pl.semaphore_signal(
                barrier_sem, inc=1,
                device_id=(nbr,), device_id_type=pl.DeviceIdType.MESH,
            )
        pl.semaphore_wait(barrier_sem, 2)

        def xfer(phase, q, target):
            return pltpu.make_async_remote_copy(
                src_ref=stage_ref.at[phase, q],
                dst_ref=comm_ref.at[phase, q],
                send_sem=send_sems.at[phase, q],
                recv_sem=recv_sems.at[phase, q],
                device_id=(target,),
                device_id_type=pl.DeviceIdType.MESH,
            )

        def p1_target(q):
            return pa if q < Q // 2 else pb

        def p2_target(q):
            return pb if q < Q // 2 else pa

        a16 = a_ref[...].astype(jnp.bfloat16)
        b16 = b_ref[...].astype(jnp.bfloat16)

        order = [q for pair in zip(range(Q // 2), range(Q // 2, Q)) for q in pair]
        partials = [None] * Q
        p1 = [None] * Q
        for q in order:
            partials[q] = jnp.dot(
                a16[q * rows:(q + 1) * rows], b16,
                preferred_element_type=jnp.float32,
            )
            stage_ref[0, q] = partials[q].astype(jnp.bfloat16)
            p1[q] = xfer(0, q, p1_target(q))
            p1[q].start()

        p2 = [None] * Q
        for q in order:
            p1[q].wait_recv()
            partials[q] = partials[q] + comm_ref[0, q].astype(jnp.float32)
            stage_ref[1, q] = partials[q].astype(jnp.bfloat16)
            p2[q] = xfer(1, q, p2_target(q))
            p2[q].start()

        for q in order:
            p2[q].wait_recv()
            z = partials[q] + comm_ref[1, q].astype(jnp.float32)
            out_ref[q * rows:(q + 1) * rows] = z / (1.0 + jnp.exp(-z))

        for q in range(Q):
            p1[q].wait_send()
            p2[q].wait_send()

    return pl.pallas_call(
        body,
        out_shape=jax.ShapeDtypeStruct((m, n), jnp.float32),
        in_specs=[
            pl.BlockSpec(memory_space=pltpu.VMEM),
            pl.BlockSpec(memory_space=pltpu.VMEM),
        ],
        out_specs=pl.BlockSpec(memory_space=pltpu.VMEM),
        scratch_shapes=[
            pltpu.VMEM((2, Q, rows, n), jnp.bfloat16),
            pltpu.VMEM((2, Q, rows, n), jnp.bfloat16),
            pltpu.SemaphoreType.DMA((2, Q)),
            pltpu.SemaphoreType.DMA((2, Q)),
        ],
        compiler_params=pltpu.CompilerParams(collective_id=0),
    )(A, B)
